# baseline (speedup 1.0000x reference)
import functools

import jax
import jax.numpy as jnp
from jax import lax
from jax.experimental import pallas as pl
from jax.experimental.pallas import tpu as pltpu
from jax.experimental.pallas import tpu_sc as plsc

L = 16


def _take(v, idx):
    return v.at[idx].get(mode="promise_in_bounds")


def _cumsum16(m):
    lanes = lax.iota(jnp.int32, L)
    v = m
    for k in (1, 2, 4, 8):
        shifted = _take(v, jnp.maximum(lanes - k, 0))
        ind = jnp.minimum(jnp.maximum(lanes - (k - 1), 0), 1)
        v = v + shifted * ind
    return v


def _bcast_last(v):
    return _take(v, jnp.full((L,), L - 1, jnp.int32))


def _emb_lookup(inp_flat, weights, *, rows_per_worker, seq_len, chunk):
    n_workers = inp_flat.shape[0] // rows_per_worker
    segs_per_row = seq_len // rows_per_worker
    n_chunks = rows_per_worker // chunk
    emb_dim = weights.shape[1]
    mesh = plsc.VectorSubcoreMesh(core_axis_name="c", subcore_axis_name="s")

    @functools.partial(
        pl.kernel,
        out_type=jax.ShapeDtypeStruct((inp_flat.shape[0], emb_dim), jnp.float32),
        mesh=mesh,
        scratch_types=[
            pltpu.VMEM((seq_len,), jnp.int32),
            pltpu.VMEM((rows_per_worker,), jnp.int32),
            pltpu.VMEM((2, chunk, emb_dim), jnp.float32),
            pltpu.SemaphoreType.DMA,
            pltpu.SemaphoreType.DMA,
            pltpu.SemaphoreType.DMA,
            pltpu.SemaphoreType.DMA,
        ],
    )
    def body(inp_hbm, table_hbm, out_hbm, row_v, idx_v, buf_v, g0, g1, o0, o1):
        wid = lax.axis_index("c") * (n_workers // 2) + lax.axis_index("s")
        seg = wid % segs_per_row
        row = wid // segs_per_row
        row_start = row * seq_len

        pltpu.sync_copy(inp_hbm.at[pl.ds(row_start, seq_len)], row_v)

        def count_chunk(k, acc):
            x = row_v[pl.ds(k * L, L)]
            return acc + jnp.minimum(x, 1)

        zero = jnp.zeros((L,), jnp.int32)
        acc = lax.fori_loop(0, seg * (rows_per_worker // L), count_chunk, zero)
        base = _bcast_last(_cumsum16(acc))

        local0 = seg * rows_per_worker

        def pos_chunk(k, carry):
            x = row_v[pl.ds(local0 + k * L, L)]
            m = jnp.minimum(x, 1)
            cs = _cumsum16(m)
            idx_v[pl.ds(k * L, L)] = (carry + cs) * m
            return carry + _bcast_last(cs)

        lax.fori_loop(0, rows_per_worker // L, pos_chunk, base)

        out0 = wid * rows_per_worker
        gs, os_ = (g0, g1), (o0, o1)

        def start_gather(i, b):
            idx_sl = idx_v.at[pl.ds(i * chunk, chunk)]
            pltpu.async_copy(table_hbm.at[idx_sl], buf_v.at[b], gs[b])

        def wait_gather(b):
            pltpu.make_async_copy(
                table_hbm.at[pl.ds(0, chunk)], buf_v.at[b], gs[b]).wait()

        def start_out(i, b):
            pltpu.async_copy(
                buf_v.at[b], out_hbm.at[pl.ds(out0, chunk)], os_[b])

        def wait_out(b):
            pltpu.make_async_copy(
                table_hbm.at[pl.ds(0, chunk)],
                out_hbm.at[pl.ds(out0, chunk)], os_[b]).wait()

        start_gather(0, 0)
        wait_gather(0)
        start_gather(1, 1)
        start_out(0, 0)

        def outer(j, _):
            i = 2 * j + 1
            wait_gather(1)
            wait_out(0)
            start_gather(i + 1, 0)
            start_out(i, 1)
            wait_gather(0)
            wait_out(1)
            start_gather(i + 2, 1)
            start_out(i + 1, 0)
            return 0

        lax.fori_loop(0, (n_chunks - 2) // 2, outer, 0)

        wait_gather(1)
        wait_out(0)
        start_out(n_chunks - 1, 1)
        wait_out(1)

    return body(inp_flat, weights)


def kernel(input, weights):
    b, seq_len = input.shape
    inp_flat = input.reshape(-1)
    out = _emb_lookup(inp_flat, weights, rows_per_worker=(b * seq_len) // 32,
                      seq_len=seq_len, chunk=32)
    return out.reshape(b, seq_len, weights.shape[1])


# speedup vs baseline: 1.0844x; 1.0844x over previous
import functools

import jax
import jax.numpy as jnp
from jax import lax
from jax.experimental import pallas as pl
from jax.experimental.pallas import tpu as pltpu
from jax.experimental.pallas import tpu_sc as plsc

L = 16


def _take(v, idx):
    return v.at[idx].get(mode="promise_in_bounds")


def _cumsum16(m):
    lanes = lax.iota(jnp.int32, L)
    v = m
    for k in (1, 2, 4, 8):
        shifted = _take(v, jnp.maximum(lanes - k, 0))
        ind = jnp.minimum(jnp.maximum(lanes - (k - 1), 0), 1)
        v = v + shifted * ind
    return v


def _bcast_last(v):
    return _take(v, jnp.full((L,), L - 1, jnp.int32))


def _emb_lookup(inp_flat, weights, *, rows_per_worker, seq_len, chunk):
    n_workers = inp_flat.shape[0] // rows_per_worker
    segs_per_row = seq_len // rows_per_worker
    n_chunks = rows_per_worker // chunk
    emb_dim = weights.shape[1]
    mesh = plsc.VectorSubcoreMesh(core_axis_name="c", subcore_axis_name="s")

    @functools.partial(
        pl.kernel,
        out_type=jax.ShapeDtypeStruct((inp_flat.shape[0], emb_dim), jnp.float32),
        mesh=mesh,
        scratch_types=[
            pltpu.VMEM((seq_len,), jnp.int32),
            pltpu.VMEM((rows_per_worker,), jnp.int32),
            pltpu.VMEM((3, chunk, emb_dim), jnp.float32),
            pltpu.SemaphoreType.DMA,
            pltpu.SemaphoreType.DMA,
            pltpu.SemaphoreType.DMA,
            pltpu.SemaphoreType.DMA,
            pltpu.SemaphoreType.DMA,
            pltpu.SemaphoreType.DMA,
        ],
    )
    def body(inp_hbm, table_hbm, out_hbm, row_v, idx_v, buf_v, g0, g1, g2, o0, o1, o2):
        wid = lax.axis_index("c") * (n_workers // 2) + lax.axis_index("s")
        seg = wid % segs_per_row
        row = wid // segs_per_row
        row_start = row * seq_len

        pltpu.sync_copy(inp_hbm.at[pl.ds(row_start, seq_len)], row_v)

        def count_chunk(k, acc):
            x = row_v[pl.ds(k * L, L)]
            return acc + jnp.minimum(x, 1)

        zero = jnp.zeros((L,), jnp.int32)
        acc = lax.fori_loop(0, seg * (rows_per_worker // L), count_chunk, zero)
        base = _bcast_last(_cumsum16(acc))

        local0 = seg * rows_per_worker

        def pos_chunk(k, carry):
            x = row_v[pl.ds(local0 + k * L, L)]
            m = jnp.minimum(x, 1)
            cs = _cumsum16(m)
            idx_v[pl.ds(k * L, L)] = (carry + cs) * m
            return carry + _bcast_last(cs)

        lax.fori_loop(0, rows_per_worker // L, pos_chunk, base)

        out0 = wid * rows_per_worker
        gs, os_ = (g0, g1, g2), (o0, o1, o2)

        def start_gather(i, b):
            idx_sl = idx_v.at[pl.ds(i * chunk, chunk)]
            pltpu.async_copy(table_hbm.at[idx_sl], buf_v.at[b], gs[b])

        def wait_gather(b):
            pltpu.make_async_copy(
                table_hbm.at[pl.ds(0, chunk)], buf_v.at[b], gs[b]).wait()

        def start_out(i, b):
            pltpu.async_copy(
                buf_v.at[b], out_hbm.at[pl.ds(out0 + i * chunk, chunk)], os_[b])

        def wait_out(b):
            pltpu.make_async_copy(
                table_hbm.at[pl.ds(0, chunk)],
                out_hbm.at[pl.ds(out0, chunk)], os_[b]).wait()

        def slot3(fn, sel):
            for b in range(3):
                @pl.when(sel == b)
                def _():
                    fn(b)

        start_gather(0, 0)
        wait_gather(0)
        start_out(0, 0)
        start_gather(1, 1)
        wait_gather(1)
        start_out(1, 1)
        start_gather(2, 2)

        def steady(i, _):
            b = i % 3
            nb = (i + 1) % 3
            slot3(wait_gather, b)
            slot3(lambda s: start_out(i, s), b)
            slot3(wait_out, nb)
            slot3(lambda s: start_gather(i + 1, s), nb)
            return 0

        lax.fori_loop(2, n_chunks - 1, steady, 0)

        b_last = (n_chunks - 1) % 3
        slot3(wait_gather, b_last)
        slot3(lambda s: start_out(n_chunks - 1, s), b_last)
        wait_out(0)
        wait_out(1)
        wait_out(2)

    return body(inp_flat, weights)


def kernel(input, weights):
    b, seq_len = input.shape
    inp_flat = input.reshape(-1)
    out = _emb_lookup(inp_flat, weights, rows_per_worker=(b * seq_len) // 32,
                      seq_len=seq_len, chunk=32)
    return out.reshape(b, seq_len, weights.shape[1])


# speedup vs baseline: 1.0927x; 1.0077x over previous
import functools

import jax
import jax.numpy as jnp
from jax import lax
from jax.experimental import pallas as pl
from jax.experimental.pallas import tpu as pltpu
from jax.experimental.pallas import tpu_sc as plsc

L = 16


def _take(v, idx):
    return v.at[idx].get(mode="promise_in_bounds")


def _cumsum16(m):
    lanes = lax.iota(jnp.int32, L)
    v = m
    for k in (1, 2, 4, 8):
        shifted = _take(v, jnp.maximum(lanes - k, 0))
        ind = jnp.minimum(jnp.maximum(lanes - (k - 1), 0), 1)
        v = v + shifted * ind
    return v


def _bcast_last(v):
    return _take(v, jnp.full((L,), L - 1, jnp.int32))


def _emb_lookup(inp_flat, weights, *, rows_per_worker, seq_len, chunk):
    n_workers = inp_flat.shape[0] // rows_per_worker
    segs_per_row = seq_len // rows_per_worker
    n_chunks = rows_per_worker // chunk
    emb_dim = weights.shape[1]
    mesh = plsc.VectorSubcoreMesh(core_axis_name="c", subcore_axis_name="s")

    @functools.partial(
        pl.kernel,
        out_type=jax.ShapeDtypeStruct((inp_flat.shape[0], emb_dim), jnp.float32),
        mesh=mesh,
        scratch_types=[
            pltpu.VMEM((seq_len,), jnp.int32),
            pltpu.VMEM((rows_per_worker,), jnp.int32),
            pltpu.VMEM((3, chunk, emb_dim), jnp.float32),
            pltpu.SemaphoreType.DMA,
            pltpu.SemaphoreType.DMA,
            pltpu.SemaphoreType.DMA,
            pltpu.SemaphoreType.DMA,
            pltpu.SemaphoreType.DMA,
            pltpu.SemaphoreType.DMA,
        ],
    )
    def body(inp_hbm, table_hbm, out_hbm, row_v, idx_v, buf_v, g0, g1, g2, o0, o1, o2):
        wid = lax.axis_index("c") * (n_workers // 2) + lax.axis_index("s")
        seg = wid % segs_per_row
        row = wid // segs_per_row
        row_start = row * seq_len

        pltpu.sync_copy(inp_hbm.at[pl.ds(row_start, seq_len)], row_v)

        def count_chunk(k, acc):
            x = row_v[pl.ds(k * L, L)]
            return acc + jnp.minimum(x, 1)

        zero = jnp.zeros((L,), jnp.int32)
        acc = lax.fori_loop(0, seg * (rows_per_worker // L), count_chunk, zero)
        base = _bcast_last(_cumsum16(acc))

        local0 = seg * rows_per_worker

        def pos_chunk(k, carry):
            x = row_v[pl.ds(local0 + k * L, L)]
            m = jnp.minimum(x, 1)
            cs = _cumsum16(m)
            idx_v[pl.ds(k * L, L)] = (carry + cs) * m
            return carry + _bcast_last(cs)

        steps_per_chunk = chunk // L
        carry0 = lax.fori_loop(0, steps_per_chunk, pos_chunk, base)

        out0 = wid * rows_per_worker
        gs, os_ = (g0, g1, g2), (o0, o1, o2)

        def start_gather(i, b):
            idx_sl = idx_v.at[pl.ds(i * chunk, chunk)]
            pltpu.async_copy(table_hbm.at[idx_sl], buf_v.at[b], gs[b])

        def wait_gather(b):
            pltpu.make_async_copy(
                table_hbm.at[pl.ds(0, chunk)], buf_v.at[b], gs[b]).wait()

        def start_out(i, b):
            pltpu.async_copy(
                buf_v.at[b], out_hbm.at[pl.ds(out0 + i * chunk, chunk)], os_[b])

        def wait_out(b):
            pltpu.make_async_copy(
                table_hbm.at[pl.ds(0, chunk)],
                out_hbm.at[pl.ds(out0, chunk)], os_[b]).wait()

        def slot3(fn, sel):
            for b in range(3):
                @pl.when(sel == b)
                def _():
                    fn(b)

        start_gather(0, 0)
        lax.fori_loop(steps_per_chunk, rows_per_worker // L, pos_chunk, carry0)
        wait_gather(0)
        start_out(0, 0)
        start_gather(1, 1)
        wait_gather(1)
        start_out(1, 1)
        start_gather(2, 2)

        def steady(i, _):
            b = i % 3
            nb = (i + 1) % 3
            slot3(wait_gather, b)
            slot3(lambda s: start_out(i, s), b)
            slot3(wait_out, nb)
            slot3(lambda s: start_gather(i + 1, s), nb)
            return 0

        lax.fori_loop(2, n_chunks - 1, steady, 0)

        b_last = (n_chunks - 1) % 3
        slot3(wait_gather, b_last)
        slot3(lambda s: start_out(n_chunks - 1, s), b_last)
        wait_out(0)
        wait_out(1)
        wait_out(2)

    return body(inp_flat, weights)


def kernel(input, weights):
    b, seq_len = input.shape
    inp_flat = input.reshape(-1)
    out = _emb_lookup(inp_flat, weights, rows_per_worker=(b * seq_len) // 32,
                      seq_len=seq_len, chunk=32)
    return out.reshape(b, seq_len, weights.shape[1])
